# baseline (speedup 1.0000x reference)
import functools

import jax
import jax.numpy as jnp
from jax.experimental import pallas as pl
from jax.experimental.pallas import tpu as pltpu


def _batcher_pairs(n):
    pairs = []

    def merge(lo, n2, r):
        step = r * 2
        if step < n2:
            merge(lo, n2, step)
            merge(lo + r, n2, step)
            for i in range(lo + r, lo + n2 - r, step):
                pairs.append((i, i + r))
        else:
            pairs.append((lo, lo + r))

    def sort(lo, n2):
        if n2 > 1:
            m = n2 // 2
            sort(lo, m)
            sort(lo + m, m)
            merge(lo, n2, 1)

    sort(0, n)
    return pairs


def _median_network(num_slots, size, target):
    pairs = _batcher_pairs(size)
    INF = -1
    val = list(range(num_slots)) + [INF] * (size - num_slots)
    ops = []
    for (i, j) in pairs:
        a, b = val[i], val[j]
        if b == INF:
            continue
        if a == INF:
            val[i], val[j] = b, INF
            continue
        ops.append((a, b))
    out_reg = val[target]
    needed = {out_reg}
    kept = []
    for (a, b) in reversed(ops):
        mn = a in needed
        mx = b in needed
        if not (mn or mx):
            continue
        kept.append((a, b, mn, mx))
        needed.add(a)
        needed.add(b)
    kept.reverse()
    return kept, out_reg


_NUM_NODE = 25
_SLOTS = _NUM_NODE + 2
_TARGET = (_SLOTS - 1) // 2
_NET, _OUT_REG = _median_network(_SLOTS, 32, _TARGET)


def _mf_kernel(pad_ref, x_ref, o_ref, *, seq_len, num_node):
    def frame_body(f, carry):
        t = jnp.where(f == 0, 0, jnp.where(f == seq_len - 1, 2, 1))
        fp = jnp.maximum(f - 1, 0)
        fn = jnp.minimum(f + 1, seq_len - 1)

        for n in range(num_node):
            regs = [None] * _SLOTS
            regs[0] = x_ref[fp, n] + pad_ref[t, n, 0]
            regs[1] = x_ref[fn, n] + pad_ref[t, n, 1]
            for j in range(num_node):
                regs[2 + j] = x_ref[f, j] + pad_ref[t, n, 2 + j]
            for (a, b, mn, mx) in _NET:
                lo = jnp.minimum(regs[a], regs[b])
                hi = jnp.maximum(regs[a], regs[b])
                if mn:
                    regs[a] = lo
                if mx:
                    regs[b] = hi
            o_ref[f, n] = regs[_OUT_REG]
        return carry

    jax.lax.fori_loop(0, seq_len, frame_body, 0)


def _tr_in_kernel(x_ref, o_ref, *, seq_len, num_node):
    def body(q, carry):
        for sub in (0, 1):
            f = 2 * q + sub
            base = (q * num_node + 12 * sub) * 128
            row_a = x_ref[0:8, pl.ds(base, 1664)]
            row_b = x_ref[8:16, pl.ds(base, 1664)]
            for n in range(num_node):
                a = row_a[:, (n + sub) * 64:(n + sub) * 64 + 64]
                b = row_b[:, (n + sub) * 64:(n + sub) * 64 + 64]
                o_ref[f, n] = jnp.concatenate([a, b], axis=-1)
        return carry

    jax.lax.fori_loop(0, seq_len // 2, body, 0)


def _pad_table(A, num_node, slots, target):
    nbr = ((A[0] + jnp.eye(num_node, dtype=A.dtype)) > 0).astype(jnp.float32)
    prev_ok = jnp.array([0.0, 1.0, 1.0], jnp.float32)[:, None, None]
    next_ok = jnp.array([1.0, 1.0, 0.0], jnp.float32)[:, None, None]
    valid = jnp.concatenate(
        [
            jnp.broadcast_to(prev_ok, (3, num_node, 1)),
            jnp.broadcast_to(next_ok, (3, num_node, 1)),
            jnp.broadcast_to(nbr[None], (3, num_node, num_node)),
        ],
        axis=2,
    )
    k = valid.sum(axis=2).astype(jnp.int32)
    p_lo = target - (k - 1) // 2
    inv = 1.0 - valid
    inv_rank = jnp.cumsum(inv, axis=2) - inv
    inf = jnp.float32(jnp.inf)
    return jnp.where(
        valid > 0,
        jnp.float32(0.0),
        jnp.where(inv_rank < p_lo[..., None].astype(jnp.float32), -inf, inf),
    )


def kernel(xs, A):
    B, S, N, D = xs.shape
    assert N == _NUM_NODE and D == 64 and S % 2 == 0 and B % 16 == 0
    lanes = 128
    rows = (B * D) // lanes
    rb = min(32, rows)
    assert rows % rb == 0 and (B * D) % lanes == 0

    pad = _pad_table(A, N, _SLOTS, _TARGET)

    xt = pl.pallas_call(
        functools.partial(_tr_in_kernel, seq_len=S, num_node=N),
        grid=(B // 16,),
        in_specs=[pl.BlockSpec((16, S * N * D), lambda i: (i, 0))],
        out_specs=pl.BlockSpec((S, N, 8, lanes), lambda i: (0, 0, i, 0)),
        out_shape=jax.ShapeDtypeStruct((S, N, rows, lanes), xs.dtype),
    )(xs.reshape(B, S * N * D))

    out = pl.pallas_call(
        functools.partial(_mf_kernel, seq_len=S, num_node=N),
        grid=(rows // rb,),
        in_specs=[
            pl.BlockSpec(memory_space=pltpu.SMEM),
            pl.BlockSpec((S, N, rb, lanes), lambda i: (0, 0, i, 0)),
        ],
        out_specs=pl.BlockSpec((S, N, rb, lanes), lambda i: (0, 0, i, 0)),
        out_shape=jax.ShapeDtypeStruct((S, N, rows, lanes), xs.dtype),
    )(pad, xt)
    return (out.reshape(S, N, B // 16, 8, 2, D)
            .transpose(2, 4, 3, 0, 1, 5)
            .reshape(B, S, N, D))


# speedup vs baseline: 1.2794x; 1.2794x over previous
import functools

import jax
import jax.numpy as jnp
from jax.experimental import pallas as pl
from jax.experimental.pallas import tpu as pltpu


def _batcher_pairs(n):
    pairs = []

    def merge(lo, n2, r):
        step = r * 2
        if step < n2:
            merge(lo, n2, step)
            merge(lo + r, n2, step)
            for i in range(lo + r, lo + n2 - r, step):
                pairs.append((i, i + r))
        else:
            pairs.append((lo, lo + r))

    def sort(lo, n2):
        if n2 > 1:
            m = n2 // 2
            sort(lo, m)
            sort(lo + m, m)
            merge(lo, n2, 1)

    sort(0, n)
    return pairs


def _median_network(num_slots, size, target):
    pairs = _batcher_pairs(size)
    INF = -1
    val = list(range(num_slots)) + [INF] * (size - num_slots)
    ops = []
    for (i, j) in pairs:
        a, b = val[i], val[j]
        if b == INF:
            continue
        if a == INF:
            val[i], val[j] = b, INF
            continue
        ops.append((a, b))
    out_reg = val[target]
    needed = {out_reg}
    kept = []
    for (a, b) in reversed(ops):
        mn = a in needed
        mx = b in needed
        if not (mn or mx):
            continue
        kept.append((a, b, mn, mx))
        needed.add(a)
        needed.add(b)
    kept.reverse()
    return kept, out_reg


_NUM_NODE = 25
_SLOTS = _NUM_NODE + 2
_TARGET = (_SLOTS - 1) // 2
_NET, _OUT_REG = _median_network(_SLOTS, 32, _TARGET)


def _mf_kernel(pad_ref, x_ref, o_ref, *, seq_len, num_node):
    def frame_body(f, carry):
        t = jnp.where(f == 0, 0, jnp.where(f == seq_len - 1, 2, 1))
        fp = jnp.maximum(f - 1, 0)
        fn = jnp.minimum(f + 1, seq_len - 1)

        for n in range(num_node):
            regs = [None] * _SLOTS
            regs[0] = x_ref[fp, n] + pad_ref[t, n, 0]
            regs[1] = x_ref[fn, n] + pad_ref[t, n, 1]
            for j in range(num_node):
                regs[2 + j] = x_ref[f, j] + pad_ref[t, n, 2 + j]
            for (a, b, mn, mx) in _NET:
                lo = jnp.minimum(regs[a], regs[b])
                hi = jnp.maximum(regs[a], regs[b])
                if mn:
                    regs[a] = lo
                if mx:
                    regs[b] = hi
            o_ref[f, n] = regs[_OUT_REG]
        return carry

    jax.lax.fori_loop(0, seq_len, frame_body, 0)


def _pad_table(A, num_node, slots, target):
    nbr = ((A[0] + jnp.eye(num_node, dtype=A.dtype)) > 0).astype(jnp.float32)
    prev_ok = jnp.array([0.0, 1.0, 1.0], jnp.float32)[:, None, None]
    next_ok = jnp.array([1.0, 1.0, 0.0], jnp.float32)[:, None, None]
    valid = jnp.concatenate(
        [
            jnp.broadcast_to(prev_ok, (3, num_node, 1)),
            jnp.broadcast_to(next_ok, (3, num_node, 1)),
            jnp.broadcast_to(nbr[None], (3, num_node, num_node)),
        ],
        axis=2,
    )
    k = valid.sum(axis=2).astype(jnp.int32)
    p_lo = target - (k - 1) // 2
    inv = 1.0 - valid
    inv_rank = jnp.cumsum(inv, axis=2) - inv
    inf = jnp.float32(jnp.inf)
    return jnp.where(
        valid > 0,
        jnp.float32(0.0),
        jnp.where(inv_rank < p_lo[..., None].astype(jnp.float32), -inf, inf),
    )


def kernel(xs, A):
    B, S, N, D = xs.shape
    assert N == _NUM_NODE
    lanes = 128
    rows = (B * D) // lanes
    rb = min(32, rows)
    assert rows % rb == 0 and (B * D) % lanes == 0

    pad = _pad_table(A, N, _SLOTS, _TARGET)
    xt = xs.transpose(1, 2, 0, 3).reshape(S, N, rows, lanes)

    out = pl.pallas_call(
        functools.partial(_mf_kernel, seq_len=S, num_node=N),
        grid=(rows // rb,),
        in_specs=[
            pl.BlockSpec(memory_space=pltpu.SMEM),
            pl.BlockSpec((S, N, rb, lanes), lambda i: (0, 0, i, 0)),
        ],
        out_specs=pl.BlockSpec((S, N, rb, lanes), lambda i: (0, 0, i, 0)),
        out_shape=jax.ShapeDtypeStruct((S, N, rows, lanes), xs.dtype),
    )(pad, xt)
    return out.reshape(S, N, B, D).transpose(2, 0, 1, 3)


# speedup vs baseline: 1.2870x; 1.0059x over previous
import functools

import jax
import jax.numpy as jnp
from jax.experimental import pallas as pl
from jax.experimental.pallas import tpu as pltpu


def _batcher_pairs(n):
    pairs = []

    def merge(lo, n2, r):
        step = r * 2
        if step < n2:
            merge(lo, n2, step)
            merge(lo + r, n2, step)
            for i in range(lo + r, lo + n2 - r, step):
                pairs.append((i, i + r))
        else:
            pairs.append((lo, lo + r))

    def sort(lo, n2):
        if n2 > 1:
            m = n2 // 2
            sort(lo, m)
            sort(lo + m, m)
            merge(lo, n2, 1)

    sort(0, n)
    return pairs


def _median_network(num_slots, size, target):
    pairs = _batcher_pairs(size)
    INF = -1
    val = list(range(num_slots)) + [INF] * (size - num_slots)
    ops = []
    for (i, j) in pairs:
        a, b = val[i], val[j]
        if b == INF:
            continue
        if a == INF:
            val[i], val[j] = b, INF
            continue
        ops.append((a, b))
    out_reg = val[target]
    needed = {out_reg}
    kept = []
    for (a, b) in reversed(ops):
        mn = a in needed
        mx = b in needed
        if not (mn or mx):
            continue
        kept.append((a, b, mn, mx))
        needed.add(a)
        needed.add(b)
    kept.reverse()
    return kept, out_reg


_NUM_NODE = 25
_SLOTS = _NUM_NODE + 2
_TARGET = (_SLOTS - 1) // 2
_NET, _OUT_REG = _median_network(_SLOTS, 32, _TARGET)


def _mf_kernel(pad_ref, x_ref, o_ref, *, seq_len, num_node):
    def frame_body(f, carry):
        t = jnp.where(f == 0, 0, jnp.where(f == seq_len - 1, 2, 1))
        fp = jnp.maximum(f - 1, 0)
        fn = jnp.minimum(f + 1, seq_len - 1)

        for n in range(num_node):
            regs = [None] * _SLOTS
            regs[0] = x_ref[fp, n] + pad_ref[t, n, 0]
            regs[1] = x_ref[fn, n] + pad_ref[t, n, 1]
            for j in range(num_node):
                if j == n:
                    regs[2 + j] = x_ref[f, j]
                else:
                    regs[2 + j] = x_ref[f, j] + pad_ref[t, n, 2 + j]
            for (a, b, mn, mx) in _NET:
                lo = jnp.minimum(regs[a], regs[b])
                hi = jnp.maximum(regs[a], regs[b])
                if mn:
                    regs[a] = lo
                if mx:
                    regs[b] = hi
            o_ref[f, n] = regs[_OUT_REG]
        return carry

    jax.lax.fori_loop(0, seq_len, frame_body, 0)


def _pad_table(A, num_node, slots, target):
    nbr = ((A[0] + jnp.eye(num_node, dtype=A.dtype)) > 0).astype(jnp.float32)
    prev_ok = jnp.array([0.0, 1.0, 1.0], jnp.float32)[:, None, None]
    next_ok = jnp.array([1.0, 1.0, 0.0], jnp.float32)[:, None, None]
    valid = jnp.concatenate(
        [
            jnp.broadcast_to(prev_ok, (3, num_node, 1)),
            jnp.broadcast_to(next_ok, (3, num_node, 1)),
            jnp.broadcast_to(nbr[None], (3, num_node, num_node)),
        ],
        axis=2,
    )
    k = valid.sum(axis=2).astype(jnp.int32)
    p_lo = target - (k - 1) // 2
    inv = 1.0 - valid
    inv_rank = jnp.cumsum(inv, axis=2) - inv
    inf = jnp.float32(jnp.inf)
    return jnp.where(
        valid > 0,
        jnp.float32(0.0),
        jnp.where(inv_rank < p_lo[..., None].astype(jnp.float32), -inf, inf),
    )


def kernel(xs, A):
    B, S, N, D = xs.shape
    assert N == _NUM_NODE
    lanes = 128
    rows = (B * D) // lanes
    rb = min(32, rows)
    assert rows % rb == 0 and (B * D) % lanes == 0

    pad = _pad_table(A, N, _SLOTS, _TARGET)
    xt = xs.transpose(1, 2, 0, 3).reshape(S, N, rows, lanes)

    out = pl.pallas_call(
        functools.partial(_mf_kernel, seq_len=S, num_node=N),
        grid=(rows // rb,),
        in_specs=[
            pl.BlockSpec(memory_space=pltpu.SMEM),
            pl.BlockSpec((S, N, rb, lanes), lambda i: (0, 0, i, 0)),
        ],
        out_specs=pl.BlockSpec((S, N, rb, lanes), lambda i: (0, 0, i, 0)),
        out_shape=jax.ShapeDtypeStruct((S, N, rows, lanes), xs.dtype),
    )(pad, xt)
    return out.reshape(S, N, B, D).transpose(2, 0, 1, 3)


# speedup vs baseline: 1.2908x; 1.0030x over previous
import functools

import jax
import jax.numpy as jnp
from jax.experimental import pallas as pl
from jax.experimental.pallas import tpu as pltpu


def _batcher_pairs(n):
    pairs = []

    def merge(lo, n2, r):
        step = r * 2
        if step < n2:
            merge(lo, n2, step)
            merge(lo + r, n2, step)
            for i in range(lo + r, lo + n2 - r, step):
                pairs.append((i, i + r))
        else:
            pairs.append((lo, lo + r))

    def sort(lo, n2):
        if n2 > 1:
            m = n2 // 2
            sort(lo, m)
            sort(lo + m, m)
            merge(lo, n2, 1)

    sort(0, n)
    return pairs


def _median_network(num_slots, size, target):
    pairs = _batcher_pairs(size)
    INF = -1
    val = list(range(num_slots)) + [INF] * (size - num_slots)
    ops = []
    for (i, j) in pairs:
        a, b = val[i], val[j]
        if b == INF:
            continue
        if a == INF:
            val[i], val[j] = b, INF
            continue
        ops.append((a, b))
    out_reg = val[target]
    needed = {out_reg}
    kept = []
    for (a, b) in reversed(ops):
        mn = a in needed
        mx = b in needed
        if not (mn or mx):
            continue
        kept.append((a, b, mn, mx))
        needed.add(a)
        needed.add(b)
    kept.reverse()
    return kept, out_reg


_NUM_NODE = 25
_SLOTS = _NUM_NODE + 2
_TARGET = (_SLOTS - 1) // 2
_NET, _OUT_REG = _median_network(_SLOTS, 32, _TARGET)


def _mf_kernel(pad_ref, x_ref, o_ref, *, seq_len, num_node):
    def emit(f, fp, fn, t):
        for n in range(num_node):
            regs = [None] * _SLOTS
            regs[0] = x_ref[fp, n]
            if t == 0:
                regs[0] = regs[0] + pad_ref[t, n, 0]
            regs[1] = x_ref[fn, n]
            if t == 2:
                regs[1] = regs[1] + pad_ref[t, n, 1]
            for j in range(num_node):
                if j == n:
                    regs[2 + j] = x_ref[f, j]
                else:
                    regs[2 + j] = x_ref[f, j] + pad_ref[t, n, 2 + j]
            for (a, b, mn, mx) in _NET:
                lo = jnp.minimum(regs[a], regs[b])
                hi = jnp.maximum(regs[a], regs[b])
                if mn:
                    regs[a] = lo
                if mx:
                    regs[b] = hi
            o_ref[f, n] = regs[_OUT_REG]

    emit(0, 0, 1, 0)

    def frame_body(f, carry):
        emit(f, f - 1, f + 1, 1)
        return carry

    jax.lax.fori_loop(1, seq_len - 1, frame_body, 0)
    emit(seq_len - 1, seq_len - 2, seq_len - 1, 2)


def _pad_table(A, num_node, slots, target):
    nbr = ((A[0] + jnp.eye(num_node, dtype=A.dtype)) > 0).astype(jnp.float32)
    prev_ok = jnp.array([0.0, 1.0, 1.0], jnp.float32)[:, None, None]
    next_ok = jnp.array([1.0, 1.0, 0.0], jnp.float32)[:, None, None]
    valid = jnp.concatenate(
        [
            jnp.broadcast_to(prev_ok, (3, num_node, 1)),
            jnp.broadcast_to(next_ok, (3, num_node, 1)),
            jnp.broadcast_to(nbr[None], (3, num_node, num_node)),
        ],
        axis=2,
    )
    k = valid.sum(axis=2).astype(jnp.int32)
    p_lo = target - (k - 1) // 2
    inv = 1.0 - valid
    inv_rank = jnp.cumsum(inv, axis=2) - inv
    inf = jnp.float32(jnp.inf)
    return jnp.where(
        valid > 0,
        jnp.float32(0.0),
        jnp.where(inv_rank < p_lo[..., None].astype(jnp.float32), -inf, inf),
    )


def kernel(xs, A):
    B, S, N, D = xs.shape
    assert N == _NUM_NODE
    lanes = 128
    rows = (B * D) // lanes
    rb = min(32, rows)
    assert rows % rb == 0 and (B * D) % lanes == 0

    pad = _pad_table(A, N, _SLOTS, _TARGET)
    xt = xs.transpose(1, 2, 0, 3).reshape(S, N, rows, lanes)

    out = pl.pallas_call(
        functools.partial(_mf_kernel, seq_len=S, num_node=N),
        grid=(rows // rb,),
        in_specs=[
            pl.BlockSpec(memory_space=pltpu.SMEM),
            pl.BlockSpec((S, N, rb, lanes), lambda i: (0, 0, i, 0)),
        ],
        out_specs=pl.BlockSpec((S, N, rb, lanes), lambda i: (0, 0, i, 0)),
        out_shape=jax.ShapeDtypeStruct((S, N, rows, lanes), xs.dtype),
    )(pad, xt)
    return out.reshape(S, N, B, D).transpose(2, 0, 1, 3)
